# baseline (speedup 1.0000x reference)
import functools

import jax
import jax.numpy as jnp
import numpy as np
from jax import lax
from jax.experimental import pallas as pl
from jax.experimental.pallas import tpu as pltpu
from jax.experimental.pallas import tpu_sc as plsc

_N = 10000
_E = 320000
_INF = 128
_OUTF = 128
_HID = 256
_H = 8
_DH = 64

_NW = 32
_CK = 5
_EC = _E // _CK
_EWC = _EC // _NW
_CH = 80
_NITC = _EWC // _CH
_NP = 10240
_NR = _NP // 16

_DST_D = 384
_SRC_D = 640
_CW = 640

_f32 = jnp.float32
_bf16 = jnp.bfloat16
_u32 = jnp.uint32


def _silu(t):
    return t * jax.nn.sigmoid(t)


def _dot(a, b):
    return jnp.dot(a, b, preferred_element_type=_f32)


def _pack(a, b):
    ua = lax.bitcast_convert_type(a.astype(_bf16).astype(_f32), _u32)
    ub = lax.bitcast_convert_type(b.astype(_bf16).astype(_f32), _u32)
    return ua | (ub >> 16)


def _hi(xu):
    return lax.bitcast_convert_type(xu & _u32(0xFFFF0000), _f32)


def _lo(xu):
    return lax.bitcast_convert_type(xu << 16, _f32)


def _prep_body(h_ref, x_ref, wq, bq, wk, bk, wv, bv,
               wc1, bc1, wc2, bc2, wc3, bc3,
               dst_ref, src_ref, coord_ref):
    h = h_ref[...].astype(_bf16)
    x = x_ref[...]
    xb = x.astype(_bf16).astype(_f32)
    hf = h_ref[...]
    q = _dot(h, wq[...]) + bq[...]
    dst_ref[:, 0:128] = _pack(q[:, 0:128], q[:, 384:512])
    dst_ref[:, 128:256] = _pack(q[:, 128:256], hf)
    dst_ref[:, 256:384] = _pack(q[:, 256:384], xb)
    k = _dot(h, wk[...]) + bk[...]
    v = _dot(h, wv[...]) + bv[...]
    src_ref[:, 0:384] = _pack(k[:, 0:384], v[:, 128:512])
    src_ref[:, 384:512] = _pack(k[:, 384:512], hf)
    src_ref[:, 512:640] = _pack(v[:, 0:128], xb)
    c1 = _silu(_dot(x, wc1[...]) + bc1[...])
    c2 = _silu(_dot(c1, wc2[...]) + bc2[...])
    coord_ref[...] = _dot(c2, wc3[...]) + bc3[...]


def _pad_rows(w, dt=_bf16):
    out = jnp.zeros((128, w.shape[1]), _f32)
    return out.at[0:w.shape[0]].set(w).astype(dt)


def _run_prep(h, x128, p):
    R = 2000
    grid = (_N // R,)
    full = lambda a: pl.BlockSpec(a.shape, lambda i: tuple(0 for _ in a.shape))
    row_spec = lambda d: pl.BlockSpec((R, d), lambda i: (i, 0))
    bf = lambda a: a.astype(_bf16)
    args = [h, x128,
            bf(p['Wq']), p['bq'].reshape(1, -1),
            bf(p['Wk']), p['bk'].reshape(1, -1),
            bf(p['Wv']), p['bv'].reshape(1, -1),
            _pad_rows(p['Wc1'], _f32), p['bc1'].reshape(1, -1),
            p['Wc2'], p['bc2'].reshape(1, -1),
            p['Wc3'], p['bc3'].reshape(1, -1)]
    in_specs = [row_spec(_INF), row_spec(128)] + [full(a) for a in args[2:]]
    return pl.pallas_call(
        _prep_body,
        grid=grid,
        in_specs=in_specs,
        out_specs=[row_spec(_DST_D), row_spec(_SRC_D), row_spec(_OUTF)],
        out_shape=[jax.ShapeDtypeStruct((_N, _DST_D), _u32),
                   jax.ShapeDtypeStruct((_N, _SRC_D), _u32),
                   jax.ShapeDtypeStruct((_N, _OUTF), _f32)],
    )(*args)


def _make_gather(D):
    mesh = plsc.VectorSubcoreMesh(core_axis_name="c", subcore_axis_name="s")

    @functools.partial(
        pl.kernel, mesh=mesh,
        out_type=jax.ShapeDtypeStruct((_EC, D), _u32),
        scratch_types=[pltpu.VMEM((_CH,), jnp.int32),
                       pltpu.VMEM((_CH,), jnp.int32),
                       pltpu.VMEM((_CH, D), _u32),
                       pltpu.VMEM((_CH, D), _u32),
                       pltpu.SemaphoreType.DMA,
                       pltpu.SemaphoreType.DMA],
    )
    def gk(table_hbm, idx_hbm, out_hbm, i0, i1, b0, b1, s0, s1):
        w = lax.axis_index("s") * 2 + lax.axis_index("c")
        base = w * _EWC

        def start(iv, bv, sv, j):
            pltpu.sync_copy(idx_hbm.at[pl.ds(base + j * _CH, _CH)], iv)
            return pltpu.async_copy(table_hbm.at[iv], bv, sv)

        def drain(bv, sv, j):
            pltpu.make_async_copy(table_hbm.at[pl.ds(0, _CH)], bv, sv).wait()
            pltpu.sync_copy(bv, out_hbm.at[pl.ds(base + j * _CH, _CH)])

        start(i0, b0, s0, 0)

        def body(t, carry):
            j0 = 2 * t
            start(i1, b1, s1, j0 + 1)
            drain(b0, s0, j0)
            start(i0, b0, s0, j0 + 2)
            drain(b1, s1, j0 + 1)
            return carry

        lax.fori_loop(0, (_NITC - 1) // 2, body, 0)
        drain(b0, s0, _NITC - 1)

    return gk


_gather_dst = _make_gather(_DST_D)
_gather_src = _make_gather(_SRC_D)


def _edge_body(dst_ref, src_ref,
               w_rd, w1a, w1b, w1xr, w1xc, be1,
               we2, be2, wew, bew, wp1r, wp1c, bp1, wp2, bp2,
               shead, expand, hmask, out_ref):
    xd = dst_ref[...]
    hi_d = _hi(xd)
    lo_d = _lo(xd)
    q = jnp.concatenate([hi_d, lo_d[:, 0:128]], axis=1)
    h_r = lo_d[:, 128:256]
    xr = lo_d[:, 256:384]
    xs = src_ref[...]
    hi_s = _hi(xs)
    lo_s = _lo(xs)
    k = hi_s[:, 0:512]
    v = jnp.concatenate([hi_s[:, 512:640], lo_s[:, 0:384]], axis=1)
    h_c = lo_s[:, 384:512]
    xc = lo_s[:, 512:640]

    d = xr - xc
    rd = jnp.sum(d * d, axis=1, keepdims=True)
    pre1 = (_dot(h_r.astype(_bf16), w1a[...])
            + _dot(h_c.astype(_bf16), w1b[...])
            + _dot(xr.astype(_bf16), w1xr[...])
            + _dot(xc.astype(_bf16), w1xc[...])
            + be1[...] + rd * w_rd[...])
    ef = _dot(_silu(pre1).astype(_bf16), we2[...]) + be2[...]
    ew = _dot(ef.astype(_bf16), wew[...]) + bew[...]
    pp = (_dot(xr.astype(_bf16), wp1r[...])
          + _dot(xc.astype(_bf16), wp1c[...]) + bp1[...])
    pos = _dot(_silu(pp).astype(_bf16), wp2[...]) + bp2[...]
    qk = (q * k).astype(_bf16)
    s = _dot(qk, shead[...]) + pos + ew
    e = jnp.exp(s) * hmask[...]
    out_ref[:, 0:512] = v * _dot(e.astype(_bf16), expand[...])
    out_ref[:, 512:528] = e
    out_ref[:, 528:640] = jnp.zeros_like(out_ref[:, 528:640])


def _run_edge(gd, gs, p):
    B = 512
    grid = (_EC // B,)
    full = lambda a: pl.BlockSpec(a.shape, lambda i: tuple(0 for _ in a.shape))
    row_spec = lambda d: pl.BlockSpec((B, d), lambda i: (i, 0))

    shead = np.zeros((512, 16), np.float32)
    for dd in range(512):
        shead[dd, dd // _DH] = 1.0 / np.sqrt(_DH)
    expand = np.zeros((16, 512), np.float32)
    for dd in range(512):
        expand[dd // _DH, dd] = 1.0
    hmask = np.zeros((1, 16), np.float32)
    hmask[0, :_H] = 1.0

    pad16 = lambda w: jnp.zeros((w.shape[0], 16), _f32).at[:, :_H].set(w).astype(_bf16)
    padb16 = lambda b: jnp.zeros((1, 16), _f32).at[0, :_H].set(b)
    wp1 = p['Wp1']
    we1 = p['We1']
    args = [gd, gs,
            we1[256].reshape(1, -1),
            we1[0:128].astype(_bf16), we1[128:256].astype(_bf16),
            _pad_rows(we1[257:260]), _pad_rows(we1[260:263]),
            p['be1'].reshape(1, -1),
            p['We2'].astype(_bf16), p['be2'].reshape(1, -1),
            pad16(p['Wew']), padb16(p['bew']),
            _pad_rows(wp1[0:3]), _pad_rows(wp1[3:6]),
            p['bp1'].reshape(1, -1),
            pad16(p['Wp2']), padb16(p['bp2']),
            jnp.asarray(shead, _bf16), jnp.asarray(expand, _bf16),
            jnp.asarray(hmask)]
    in_specs = [row_spec(_DST_D), row_spec(_SRC_D)]
    in_specs += [full(a) for a in args[2:]]
    return pl.pallas_call(
        _edge_body,
        grid=grid,
        in_specs=in_specs,
        out_specs=row_spec(_CW),
        out_shape=jax.ShapeDtypeStruct((_EC, _CW), _f32),
    )(*args)


def _make_scatter(c0, chunk_ids):
    mesh = plsc.VectorSubcoreMesh(core_axis_name="c", subcore_axis_name="s")
    ncs = len(chunk_ids)

    @functools.partial(
        pl.kernel, mesh=mesh,
        out_type=jax.ShapeDtypeStruct((2, _NP, 128), _f32),
        scratch_types=[pltpu.VMEM((_CH,), jnp.int32),
                       pltpu.VMEM((_CH,), jnp.int32),
                       pltpu.VMEM((_CH, 128), _f32),
                       pltpu.VMEM((_CH, 128), _f32),
                       pltpu.VMEM_SHARED((_NP, 128), _f32),
                       pltpu.SemaphoreType.DMA,
                       pltpu.SemaphoreType.DMA],
    )
    def sk(*refs):
        chbms = refs[:ncs]
        row_hbm, zeros_hbm, out_hbm, i0, i1, d0, d1, acc_sh, s0, s1 = refs[ncs:]
        ci = lax.axis_index("c")
        si = lax.axis_index("s")
        w = si * 2 + ci
        base = w * _EWC
        pltpu.sync_copy(zeros_hbm, acc_sh.at[pl.ds(si * _NR, _NR)])
        plsc.subcore_barrier()

        for ch, chbm in zip(chunk_ids, chbms):
            gbase = ch * _EC + base

            def start(iv, dv, sv, j, chbm=chbm, gbase=gbase):
                pltpu.sync_copy(row_hbm.at[pl.ds(gbase + j * _CH, _CH)], iv)
                return pltpu.async_copy(
                    chbm.at[pl.ds(base + j * _CH, _CH), pl.ds(c0, 128)],
                    dv, sv)

            def reduce(iv, dv, sv, chbm=chbm):
                pltpu.make_async_copy(
                    chbm.at[pl.ds(0, _CH), pl.ds(c0, 128)], dv, sv).wait()
                pltpu.sync_copy(dv, acc_sh.at[iv], add=True)

            start(i0, d0, s0, 0)

            def body(t, carry, start=start, reduce=reduce):
                j0 = 2 * t
                start(i1, d1, s1, j0 + 1)
                reduce(i0, d0, s0)
                start(i0, d0, s0, j0 + 2)
                reduce(i1, d1, s1)
                return carry

            lax.fori_loop(0, (_NITC - 1) // 2, body, 0)
            reduce(i0, d0, s0)

        plsc.subcore_barrier()
        pltpu.sync_copy(acc_sh.at[pl.ds(si * _NR, _NR)],
                        out_hbm.at[ci, pl.ds(si * _NR, _NR)])

    return sk


_CGROUPS = ((0, 1), (2, 3), (4,))
_scatter_groups = [
    [_make_scatter(c0, g) for c0 in (0, 128, 256, 384, 512)]
    for g in _CGROUPS
]


def _final_body(*refs):
    parts = refs[:15]
    coord_ref, wo0, wo1, wo2, wo3, bo, e0, e1, e2, e3, out_ref = refs[15:]

    def tot(c):
        t = parts[c][0] + parts[c][1]
        for g in (1, 2):
            pg = parts[g * 5 + c]
            t = t + pg[0] + pg[1]
        return t

    es = tot(4)[:, 0:16]
    rec = 1.0 / (es + 1e-8)
    acc = coord_ref[...] + bo[...]
    for c, (woc, ec) in enumerate(((wo0, e0), (wo1, e1),
                                   (wo2, e2), (wo3, e3))):
        wvc = tot(c)
        norm = wvc * _dot(rec, ec[...])
        acc = acc + _dot(norm, woc[...])
    out_ref[...] = acc


def _run_final(parts, coord, p):
    R = 1000
    grid = (_N // R,)
    full = lambda a: pl.BlockSpec(a.shape, lambda i: tuple(0 for _ in a.shape))
    pspec = pl.BlockSpec((2, R, 128), lambda i: (0, i, 0))

    expand = np.zeros((16, 512), np.float32)
    for dd in range(512):
        expand[dd // _DH, dd] = 1.0
    wo = p['Wo']
    args = list(parts) + [coord]
    args += [wo[c * 128:(c + 1) * 128] for c in range(4)]
    args += [p['bo'].reshape(1, -1)]
    args += [jnp.asarray(expand[:, c * 128:(c + 1) * 128]) for c in range(4)]
    in_specs = [pspec] * 15 + [pl.BlockSpec((R, 128), lambda i: (i, 0))]
    in_specs += [full(a) for a in args[16:]]
    return pl.pallas_call(
        _final_body,
        grid=grid,
        in_specs=in_specs,
        out_specs=pl.BlockSpec((R, _OUTF), lambda i: (i, 0)),
        out_shape=jax.ShapeDtypeStruct((_N, _OUTF), _f32),
    )(*args)


def kernel(h, x, edge_index, params):
    p = params
    row = edge_index[0].astype(jnp.int32)
    col = edge_index[1].astype(jnp.int32)
    x128 = jnp.pad(x.astype(_f32), ((0, 0), (0, 125)))

    dst_t, src_t, coord = _run_prep(h, x128, p)

    contribs = []
    for c in range(_CK):
        rowc = lax.dynamic_slice_in_dim(row, c * _EC, _EC)
        colc = lax.dynamic_slice_in_dim(col, c * _EC, _EC)
        gd = _gather_dst(dst_t, rowc)
        gs = _gather_src(src_t, colc)
        contribs.append(_run_edge(gd, gs, p))

    z128 = jnp.zeros((_NR, 128), _f32)
    parts = []
    for g, sks in zip(_CGROUPS, _scatter_groups):
        parts += [sk(*[contribs[i] for i in g], row, z128) for sk in sks]

    out = _run_final(parts, coord, p)
    return (out, jnp.zeros_like(x))


# speedup vs baseline: 1.0424x; 1.0424x over previous
import functools

import jax
import jax.numpy as jnp
import numpy as np
from jax import lax
from jax.experimental import pallas as pl
from jax.experimental.pallas import tpu as pltpu
from jax.experimental.pallas import tpu_sc as plsc

_N = 10000
_E = 320000
_INF = 128
_OUTF = 128
_HID = 256
_H = 8
_DH = 64

_NW = 32
_CK = 5
_EC = _E // _CK
_EWC = _EC // _NW
_CH = 80
_NITC = _EWC // _CH
_NP = 10240
_NR = _NP // 16

_DST_D = 384
_SRC_D = 640
_CW = 640

_f32 = jnp.float32
_bf16 = jnp.bfloat16
_u32 = jnp.uint32


def _silu(t):
    return t * jax.nn.sigmoid(t)


def _dot(a, b):
    return jnp.dot(a, b, preferred_element_type=_f32)


def _pack(a, b):
    ua = lax.bitcast_convert_type(a.astype(_bf16).astype(_f32), _u32)
    ub = lax.bitcast_convert_type(b.astype(_bf16).astype(_f32), _u32)
    return ua | (ub >> 16)


def _hi(xu):
    return lax.bitcast_convert_type(xu & _u32(0xFFFF0000), _f32)


def _lo(xu):
    return lax.bitcast_convert_type(xu << 16, _f32)


def _prep_body(h_ref, x_ref, wq, bq, wk, bk, wv, bv,
               wc1, bc1, wc2, bc2, wc3, bc3,
               dst_ref, src_ref, coord_ref):
    h = h_ref[...].astype(_bf16)
    x = x_ref[...]
    xb = x.astype(_bf16).astype(_f32)
    hf = h_ref[...]
    q = _dot(h, wq[...]) + bq[...]
    dst_ref[:, 0:128] = _pack(q[:, 0:128], q[:, 384:512])
    dst_ref[:, 128:256] = _pack(q[:, 128:256], hf)
    dst_ref[:, 256:384] = _pack(q[:, 256:384], xb)
    k = _dot(h, wk[...]) + bk[...]
    v = _dot(h, wv[...]) + bv[...]
    src_ref[:, 0:384] = _pack(k[:, 0:384], v[:, 128:512])
    src_ref[:, 384:512] = _pack(k[:, 384:512], hf)
    src_ref[:, 512:640] = _pack(v[:, 0:128], xb)
    c1 = _silu(_dot(x, wc1[...]) + bc1[...])
    c2 = _silu(_dot(c1, wc2[...]) + bc2[...])
    coord_ref[...] = _dot(c2, wc3[...]) + bc3[...]


def _pad_rows(w, dt=_bf16):
    out = jnp.zeros((128, w.shape[1]), _f32)
    return out.at[0:w.shape[0]].set(w).astype(dt)


def _run_prep(h, x128, p):
    R = 2000
    grid = (_N // R,)
    full = lambda a: pl.BlockSpec(a.shape, lambda i: tuple(0 for _ in a.shape))
    row_spec = lambda d: pl.BlockSpec((R, d), lambda i: (i, 0))
    bf = lambda a: a.astype(_bf16)
    args = [h, x128,
            bf(p['Wq']), p['bq'].reshape(1, -1),
            bf(p['Wk']), p['bk'].reshape(1, -1),
            bf(p['Wv']), p['bv'].reshape(1, -1),
            _pad_rows(p['Wc1'], _f32), p['bc1'].reshape(1, -1),
            p['Wc2'], p['bc2'].reshape(1, -1),
            p['Wc3'], p['bc3'].reshape(1, -1)]
    in_specs = [row_spec(_INF), row_spec(128)] + [full(a) for a in args[2:]]
    return pl.pallas_call(
        _prep_body,
        grid=grid,
        in_specs=in_specs,
        out_specs=[row_spec(_DST_D), row_spec(_SRC_D), row_spec(_OUTF)],
        out_shape=[jax.ShapeDtypeStruct((_N, _DST_D), _u32),
                   jax.ShapeDtypeStruct((_N, _SRC_D), _u32),
                   jax.ShapeDtypeStruct((_N, _OUTF), _f32)],
    )(*args)


def _make_gather(D):
    mesh = plsc.VectorSubcoreMesh(core_axis_name="c", subcore_axis_name="s")

    @functools.partial(
        pl.kernel, mesh=mesh,
        out_type=jax.ShapeDtypeStruct((_EC, D), _u32),
        scratch_types=[pltpu.VMEM((_CH,), jnp.int32),
                       pltpu.VMEM((_CH,), jnp.int32),
                       pltpu.VMEM((_CH, D), _u32),
                       pltpu.VMEM((_CH, D), _u32),
                       pltpu.SemaphoreType.DMA,
                       pltpu.SemaphoreType.DMA],
    )
    def gk(table_hbm, idx_hbm, out_hbm, i0, i1, b0, b1, s0, s1):
        w = lax.axis_index("s") * 2 + lax.axis_index("c")
        base = w * _EWC

        def start(iv, bv, sv, j):
            pltpu.sync_copy(idx_hbm.at[pl.ds(base + j * _CH, _CH)], iv)
            return pltpu.async_copy(table_hbm.at[iv], bv, sv)

        def drain(bv, sv, j):
            pltpu.make_async_copy(table_hbm.at[pl.ds(0, _CH)], bv, sv).wait()
            pltpu.sync_copy(bv, out_hbm.at[pl.ds(base + j * _CH, _CH)])

        start(i0, b0, s0, 0)

        def body(t, carry):
            j0 = 2 * t
            start(i1, b1, s1, j0 + 1)
            drain(b0, s0, j0)
            start(i0, b0, s0, j0 + 2)
            drain(b1, s1, j0 + 1)
            return carry

        lax.fori_loop(0, (_NITC - 1) // 2, body, 0)
        drain(b0, s0, _NITC - 1)

    return gk


_gather_dst = _make_gather(_DST_D)
_gather_src = _make_gather(_SRC_D)


def _edge_body(dst_ref, src_ref,
               w_rd, w1a, w1b, w1xr, w1xc, be1,
               we2, be2, wew, bew, wp1r, wp1c, bp1, wp2, bp2,
               shead, expand, hmask, out_ref):
    xd = dst_ref[...]
    hi_d = _hi(xd)
    lo_d = _lo(xd)
    q = jnp.concatenate([hi_d, lo_d[:, 0:128]], axis=1)
    h_r = lo_d[:, 128:256]
    xr = lo_d[:, 256:384]
    xs = src_ref[...]
    hi_s = _hi(xs)
    lo_s = _lo(xs)
    k = hi_s[:, 0:512]
    v = jnp.concatenate([hi_s[:, 512:640], lo_s[:, 0:384]], axis=1)
    h_c = lo_s[:, 384:512]
    xc = lo_s[:, 512:640]

    d = xr - xc
    rd = jnp.sum(d * d, axis=1, keepdims=True)
    pre1 = (_dot(h_r.astype(_bf16), w1a[...])
            + _dot(h_c.astype(_bf16), w1b[...])
            + _dot(xr.astype(_bf16), w1xr[...])
            + _dot(xc.astype(_bf16), w1xc[...])
            + be1[...] + rd * w_rd[...])
    ef = _dot(_silu(pre1).astype(_bf16), we2[...]) + be2[...]
    ew = _dot(ef.astype(_bf16), wew[...]) + bew[...]
    pp = (_dot(xr.astype(_bf16), wp1r[...])
          + _dot(xc.astype(_bf16), wp1c[...]) + bp1[...])
    pos = _dot(_silu(pp).astype(_bf16), wp2[...]) + bp2[...]
    qk = (q * k).astype(_bf16)
    s = _dot(qk, shead[...]) + pos + ew
    e = jnp.exp(s) * hmask[...]
    out_ref[:, 0:512] = v * _dot(e.astype(_bf16), expand[...])
    out_ref[:, 512:528] = e
    out_ref[:, 528:640] = jnp.zeros_like(out_ref[:, 528:640])


def _run_edge(gd, gs, p):
    B = 512
    grid = (_EC // B,)
    full = lambda a: pl.BlockSpec(a.shape, lambda i: tuple(0 for _ in a.shape))
    row_spec = lambda d: pl.BlockSpec((B, d), lambda i: (i, 0))

    shead = np.zeros((512, 16), np.float32)
    for dd in range(512):
        shead[dd, dd // _DH] = 1.0 / np.sqrt(_DH)
    expand = np.zeros((16, 512), np.float32)
    for dd in range(512):
        expand[dd // _DH, dd] = 1.0
    hmask = np.zeros((1, 16), np.float32)
    hmask[0, :_H] = 1.0

    pad16 = lambda w: jnp.zeros((w.shape[0], 16), _f32).at[:, :_H].set(w).astype(_bf16)
    padb16 = lambda b: jnp.zeros((1, 16), _f32).at[0, :_H].set(b)
    wp1 = p['Wp1']
    we1 = p['We1']
    args = [gd, gs,
            we1[256].reshape(1, -1),
            we1[0:128].astype(_bf16), we1[128:256].astype(_bf16),
            _pad_rows(we1[257:260]), _pad_rows(we1[260:263]),
            p['be1'].reshape(1, -1),
            p['We2'].astype(_bf16), p['be2'].reshape(1, -1),
            pad16(p['Wew']), padb16(p['bew']),
            _pad_rows(wp1[0:3]), _pad_rows(wp1[3:6]),
            p['bp1'].reshape(1, -1),
            pad16(p['Wp2']), padb16(p['bp2']),
            jnp.asarray(shead, _bf16), jnp.asarray(expand, _bf16),
            jnp.asarray(hmask)]
    in_specs = [row_spec(_DST_D), row_spec(_SRC_D)]
    in_specs += [full(a) for a in args[2:]]
    return pl.pallas_call(
        _edge_body,
        grid=grid,
        in_specs=in_specs,
        out_specs=row_spec(_CW),
        out_shape=jax.ShapeDtypeStruct((_EC, _CW), _f32),
    )(*args)


def _make_scatter(c0, chunk_ids):
    mesh = plsc.VectorSubcoreMesh(core_axis_name="c", subcore_axis_name="s")
    ncs = len(chunk_ids)

    @functools.partial(
        pl.kernel, mesh=mesh,
        out_type=jax.ShapeDtypeStruct((2, _NP, 128), _f32),
        scratch_types=[pltpu.VMEM((_CH,), jnp.int32),
                       pltpu.VMEM((_CH,), jnp.int32),
                       pltpu.VMEM((_CH, 128), _f32),
                       pltpu.VMEM((_CH, 128), _f32),
                       pltpu.VMEM_SHARED((_NP, 128), _f32),
                       pltpu.SemaphoreType.DMA,
                       pltpu.SemaphoreType.DMA],
    )
    def sk(*refs):
        chbms = refs[:ncs]
        row_hbm, zeros_hbm, out_hbm, i0, i1, d0, d1, acc_sh, s0, s1 = refs[ncs:]
        ci = lax.axis_index("c")
        si = lax.axis_index("s")
        w = si * 2 + ci
        base = w * _EWC
        pltpu.sync_copy(zeros_hbm, acc_sh.at[pl.ds(si * _NR, _NR)])
        plsc.subcore_barrier()

        for ch, chbm in zip(chunk_ids, chbms):
            gbase = ch * _EC + base

            def start(iv, dv, sv, j, chbm=chbm, gbase=gbase):
                pltpu.sync_copy(row_hbm.at[pl.ds(gbase + j * _CH, _CH)], iv)
                return pltpu.async_copy(
                    chbm.at[pl.ds(base + j * _CH, _CH), pl.ds(c0, 128)],
                    dv, sv)

            def reduce(iv, dv, sv, chbm=chbm):
                pltpu.make_async_copy(
                    chbm.at[pl.ds(0, _CH), pl.ds(c0, 128)], dv, sv).wait()
                pltpu.sync_copy(dv, acc_sh.at[iv], add=True)

            start(i0, d0, s0, 0)

            def body(t, carry, start=start, reduce=reduce):
                j0 = 2 * t
                start(i1, d1, s1, j0 + 1)
                reduce(i0, d0, s0)
                start(i0, d0, s0, j0 + 2)
                reduce(i1, d1, s1)
                return carry

            lax.fori_loop(0, (_NITC - 1) // 2, body, 0)
            reduce(i0, d0, s0)

        plsc.subcore_barrier()
        pltpu.sync_copy(acc_sh.at[pl.ds(si * _NR, _NR)],
                        out_hbm.at[ci, pl.ds(si * _NR, _NR)])

    return sk


_CGROUPS = ((0, 1, 2), (3, 4))
_scatter_groups = [
    [_make_scatter(c0, g) for c0 in (0, 128, 256, 384, 512)]
    for g in _CGROUPS
]


def _final_body(*refs):
    parts = refs[:10]
    coord_ref, wo0, wo1, wo2, wo3, bo, e0, e1, e2, e3, out_ref = refs[10:]

    def tot(c):
        t = parts[c][0] + parts[c][1]
        pg = parts[5 + c]
        return t + pg[0] + pg[1]

    es = tot(4)[:, 0:16]
    rec = 1.0 / (es + 1e-8)
    acc = coord_ref[...] + bo[...]
    for c, (woc, ec) in enumerate(((wo0, e0), (wo1, e1),
                                   (wo2, e2), (wo3, e3))):
        wvc = tot(c)
        norm = wvc * _dot(rec, ec[...])
        acc = acc + _dot(norm, woc[...])
    out_ref[...] = acc


def _run_final(parts, coord, p):
    R = 1000
    grid = (_N // R,)
    full = lambda a: pl.BlockSpec(a.shape, lambda i: tuple(0 for _ in a.shape))
    pspec = pl.BlockSpec((2, R, 128), lambda i: (0, i, 0))

    expand = np.zeros((16, 512), np.float32)
    for dd in range(512):
        expand[dd // _DH, dd] = 1.0
    wo = p['Wo']
    args = list(parts) + [coord]
    args += [wo[c * 128:(c + 1) * 128] for c in range(4)]
    args += [p['bo'].reshape(1, -1)]
    args += [jnp.asarray(expand[:, c * 128:(c + 1) * 128]) for c in range(4)]
    in_specs = [pspec] * 10 + [pl.BlockSpec((R, 128), lambda i: (i, 0))]
    in_specs += [full(a) for a in args[11:]]
    return pl.pallas_call(
        _final_body,
        grid=grid,
        in_specs=in_specs,
        out_specs=pl.BlockSpec((R, _OUTF), lambda i: (i, 0)),
        out_shape=jax.ShapeDtypeStruct((_N, _OUTF), _f32),
    )(*args)


def kernel(h, x, edge_index, params):
    p = params
    row = edge_index[0].astype(jnp.int32)
    col = edge_index[1].astype(jnp.int32)
    x128 = jnp.pad(x.astype(_f32), ((0, 0), (0, 125)))

    dst_t, src_t, coord = _run_prep(h, x128, p)

    contribs = []
    for c in range(_CK):
        rowc = lax.dynamic_slice_in_dim(row, c * _EC, _EC)
        colc = lax.dynamic_slice_in_dim(col, c * _EC, _EC)
        gd = _gather_dst(dst_t, rowc)
        gs = _gather_src(src_t, colc)
        contribs.append(_run_edge(gd, gs, p))

    z128 = jnp.zeros((_NR, 128), _f32)
    parts = []
    for g, sks in zip(_CGROUPS, _scatter_groups):
        parts += [sk(*[contribs[i] for i in g], row, z128) for sk in sks]

    out = _run_final(parts, coord, p)
    return (out, jnp.zeros_like(x))
